# baseline (speedup 1.0000x reference)
import functools

import jax
import jax.numpy as jnp
from jax import lax
from jax.experimental import pallas as pl
from jax.experimental.pallas import tpu as pltpu
from jax.experimental.pallas import tpu_sc as plsc

NC = 2
NS = 16
NW = NC * NS
CHUNK = 64
NBUF = 3
ROW_BLK = 1280


def _round_up(x, m):
    return (x + m - 1) // m * m


def _make_deg_kernel(np_nodes, epw):
    mesh = plsc.VectorSubcoreMesh(core_axis_name="c", subcore_axis_name="s")

    @functools.partial(
        pl.kernel,
        out_type=jax.ShapeDtypeStruct((NW, np_nodes), jnp.float32),
        mesh=mesh,
        compiler_params=pltpu.CompilerParams(needs_layout_passes=False),
        scratch_types=[
            pltpu.VMEM((epw,), jnp.int32),
            pltpu.VMEM((np_nodes,), jnp.float32),
        ],
    )
    def deg_kernel(dst_hbm, out_hbm, idx_v, hist_v):
        ci = lax.axis_index("c")
        si = lax.axis_index("s")
        w = ci * NS + si
        pltpu.sync_copy(dst_hbm.at[w], idx_v)
        z16 = jnp.zeros((16,), jnp.float32)

        def zbody(i, carry):
            hist_v[pl.ds(i * 16, 16)] = z16
            return carry

        lax.fori_loop(0, np_nodes // 16, zbody, 0)
        ones16 = jnp.ones((16,), jnp.float32)

        def body(i, carry):
            for u in range(4):
                idx16 = idx_v[pl.ds((i * 4 + u) * 16, 16)]
                plsc.addupdate_scatter(hist_v, [idx16], ones16)
            return carry

        lax.fori_loop(0, epw // 64, body, 0)
        pltpu.sync_copy(hist_v, out_hbm.at[w])

    return deg_kernel


def _make_hop_kernel(np_nodes, n, epw, c):
    mesh = plsc.VectorSubcoreMesh(core_axis_name="c", subcore_axis_name="s")
    nchunk = epw // CHUNK
    rows_per_tile = np_nodes // NS
    drows = (n // NS) // 8 * 8
    drows_last = n - (NS - 1) * drows

    @functools.partial(
        pl.kernel,
        out_type=jax.ShapeDtypeStruct((NC, n, c), jnp.float32),
        mesh=mesh,
        compiler_params=pltpu.CompilerParams(needs_layout_passes=False),
        scratch_types=[
            pltpu.VMEM((nchunk, CHUNK), jnp.int32),
            pltpu.VMEM((nchunk, CHUNK), jnp.int32),
            [pltpu.VMEM((CHUNK, c), jnp.float32)] * NBUF,
            pltpu.VMEM((16, c), jnp.float32),
            pltpu.VMEM_SHARED((np_nodes, c), jnp.float32),
            [pltpu.SemaphoreType.DMA] * NBUF,
            [pltpu.SemaphoreType.DMA] * NBUF,
        ],
    )
    def hop_kernel(table_hbm, src_hbm, dst_hbm, zeros_hbm, out_hbm,
                   src_v, dst_v, bufs, zbuf, acc, gsems, ssems):
        ci = lax.axis_index("c")
        si = lax.axis_index("s")
        w = ci * NS + si
        cz = pltpu.async_copy(zeros_hbm, zbuf, ssems[0])
        cs = pltpu.async_copy(src_hbm.at[w], src_v, gsems[0])
        cd = pltpu.async_copy(dst_hbm.at[w], dst_v, gsems[1])
        cz.wait()
        base = si * rows_per_tile
        nz = rows_per_tile // 16
        for g in range(0, nz, 8):
            zcs = [pltpu.async_copy(
                       zbuf, acc.at[pl.ds(base + (g + u) * 16, 16)],
                       ssems[1])
                   for u in range(min(8, nz - g))]
            for zc in zcs:
                zc.wait()
        cs.wait()
        cd.wait()
        plsc.subcore_barrier()

        for i in range(NBUF - 1):
            pltpu.async_copy(table_hbm.at[src_v.at[i]], bufs[i], gsems[i])

        def ring(k, carry):
            j0 = k * NBUF
            for i in range(NBUF):
                j = j0 + i
                i3 = (i + NBUF - 1) % NBUF
                jn = j + NBUF - 1

                @pl.when(jnp.logical_and(jn < nchunk, j >= 1))
                def _():
                    pltpu.make_async_copy(
                        bufs[i3], acc.at[dst_v.at[j - 1]], ssems[i3]).wait()
                    pltpu.async_copy(table_hbm.at[src_v.at[jn]],
                                     bufs[i3], gsems[i3])

                @pl.when(jnp.logical_and(jn < nchunk, j < 1))
                def _():
                    pltpu.async_copy(table_hbm.at[src_v.at[jn]],
                                     bufs[i3], gsems[i3])

                pltpu.make_async_copy(table_hbm.at[src_v.at[j]],
                                      bufs[i], gsems[i]).wait()
                pltpu.async_copy(bufs[i], acc.at[dst_v.at[j]],
                                 ssems[i], add=True)
            return carry

        lax.fori_loop(0, nchunk // NBUF, ring, 0)
        for i in range(NBUF):
            j = nchunk - NBUF + i
            pltpu.make_async_copy(bufs[j % NBUF], acc.at[dst_v.at[j]],
                                  ssems[j % NBUF]).wait()
        plsc.subcore_barrier()
        @pl.when(si < NS - 1)
        def _():
            pltpu.sync_copy(acc.at[pl.ds(si * drows, drows)],
                            out_hbm.at[ci, pl.ds(si * drows, drows)])

        @pl.when(si == NS - 1)
        def _():
            pltpu.sync_copy(
                acc.at[pl.ds((NS - 1) * drows, drows_last)],
                out_hbm.at[ci, pl.ds((NS - 1) * drows, drows_last)])

    return hop_kernel


def _mm_body(x_ref, w_ref, o_ref):
    o_ref[...] = jnp.dot(x_ref[...], w_ref[...],
                         preferred_element_type=jnp.float32)


def _scale_body(degp_ref, y_ref, t0_ref, nrm_ref):
    i = pl.program_id(0)
    deg = jnp.maximum(jnp.sum(degp_ref[...], axis=0), 1.0)
    nrm = lax.rsqrt(deg)
    t0_ref[...] = y_ref[...] * nrm[:, None]
    nrm_ref[pl.ds(i * ROW_BLK, ROW_BLK)] = nrm


def _mid_body(p_ref, nrm_ref, o_ref):
    i = pl.program_id(0)
    nrm = nrm_ref[pl.ds(i * ROW_BLK, ROW_BLK)]
    o_ref[...] = (p_ref[0] + p_ref[1]) * (nrm * nrm)[:, None]


def _fin_body(p_ref, nrm_ref, b_ref, o_ref):
    i = pl.program_id(0)
    nrm = nrm_ref[pl.ds(i * ROW_BLK, ROW_BLK)]
    o_ref[...] = ((p_ref[0] + p_ref[1]) * nrm[:, None]
                  + b_ref[...][None, :])


def kernel(features, edge_index, W, b):
    n, f = features.shape
    c = W.shape[1]
    e = edge_index.shape[1]

    epw = _round_up(_round_up(e, NW) // NW, CHUNK * NBUF)
    e_pad = NW * epw
    np_nodes = _round_up(n + 1, ROW_BLK)
    rows_per_tile = np_nodes // NS
    nblk = np_nodes // ROW_BLK
    trash = n

    src = edge_index[0].astype(jnp.int32)
    dst = edge_index[1].astype(jnp.int32)
    n_pad_edges = e_pad - e
    pad_iota = jnp.arange(n_pad_edges, dtype=jnp.int32)
    n_trash = np_nodes - trash
    src_p = jnp.concatenate([src, pad_iota % n])
    dst_p = jnp.concatenate([dst, trash + pad_iota % n_trash])
    src3 = src_p.reshape(NW, epw // CHUNK, CHUNK)
    dst3 = dst_p.reshape(NW, epw // CHUNK, CHUNK)
    dst2 = dst_p.reshape(NW, epw)

    zeros_rows = jnp.zeros((16, c), jnp.float32)

    deg_kernel = _make_deg_kernel(np_nodes, epw)
    hop_kernel = _make_hop_kernel(np_nodes, n, epw, c)

    mm = pl.pallas_call(
        _mm_body,
        grid=(nblk,),
        in_specs=[
            pl.BlockSpec((ROW_BLK, f), lambda i: (i, 0)),
            pl.BlockSpec((f, c), lambda i: (0, 0)),
        ],
        out_specs=pl.BlockSpec((ROW_BLK, c), lambda i: (i, 0)),
        out_shape=jax.ShapeDtypeStruct((n, c), jnp.float32),
    )

    normk = pl.pallas_call(
        _scale_body,
        grid=(nblk,),
        in_specs=[
            pl.BlockSpec((NW, ROW_BLK), lambda i: (0, i)),
            pl.BlockSpec((ROW_BLK, c), lambda i: (i, 0)),
        ],
        out_specs=[
            pl.BlockSpec((ROW_BLK, c), lambda i: (i, 0)),
            pl.BlockSpec((np_nodes,), lambda i: (0,)),
        ],
        out_shape=[
            jax.ShapeDtypeStruct((n, c), jnp.float32),
            jax.ShapeDtypeStruct((np_nodes,), jnp.float32),
        ],
    )

    midk = pl.pallas_call(
        _mid_body,
        grid=(nblk,),
        in_specs=[
            pl.BlockSpec((NC, ROW_BLK, c), lambda i: (0, i, 0)),
            pl.BlockSpec((np_nodes,), lambda i: (0,)),
        ],
        out_specs=pl.BlockSpec((ROW_BLK, c), lambda i: (i, 0)),
        out_shape=jax.ShapeDtypeStruct((n, c), jnp.float32),
    )

    fink = pl.pallas_call(
        _fin_body,
        grid=(nblk,),
        in_specs=[
            pl.BlockSpec((NC, ROW_BLK, c), lambda i: (0, i, 0)),
            pl.BlockSpec((np_nodes,), lambda i: (0,)),
            pl.BlockSpec((c,), lambda i: (0,)),
        ],
        out_specs=pl.BlockSpec((ROW_BLK, c), lambda i: (i, 0)),
        out_shape=jax.ShapeDtypeStruct((n, c), jnp.float32),
    )

    y = mm(features, W)
    degpart = deg_kernel(dst2)
    t0, nrm = normk(degpart, y)
    p1 = hop_kernel(t0, src3, dst3, zeros_rows)
    t1 = midk(p1, nrm)
    p2 = hop_kernel(t1, src3, dst3, zeros_rows)
    return fink(p2, nrm, b)


# speedup vs baseline: 1.0250x; 1.0250x over previous
import functools

import jax
import jax.numpy as jnp
from jax import lax
from jax.experimental import pallas as pl
from jax.experimental.pallas import tpu as pltpu
from jax.experimental.pallas import tpu_sc as plsc

NC = 2
NS = 16
NW = NC * NS
CHUNK = 64
NBUF = 3
ROW_BLK = 1280


def _round_up(x, m):
    return (x + m - 1) // m * m


def _make_deg_kernel(np_nodes, epw):
    mesh = plsc.VectorSubcoreMesh(core_axis_name="c", subcore_axis_name="s")

    @functools.partial(
        pl.kernel,
        out_type=jax.ShapeDtypeStruct((NW, np_nodes), jnp.float32),
        mesh=mesh,
        compiler_params=pltpu.CompilerParams(needs_layout_passes=False),
        scratch_types=[
            pltpu.VMEM((epw,), jnp.int32),
            pltpu.VMEM((np_nodes,), jnp.float32),
        ],
    )
    def deg_kernel(dst_hbm, out_hbm, idx_v, hist_v):
        ci = lax.axis_index("c")
        si = lax.axis_index("s")
        w = ci * NS + si
        pltpu.sync_copy(dst_hbm.at[w], idx_v)
        z16 = jnp.zeros((16,), jnp.float32)

        def zbody(i, carry):
            hist_v[pl.ds(i * 16, 16)] = z16
            return carry

        lax.fori_loop(0, np_nodes // 16, zbody, 0)
        ones16 = jnp.ones((16,), jnp.float32)

        def body(i, carry):
            for u in range(4):
                idx16 = idx_v[pl.ds((i * 4 + u) * 16, 16)]
                plsc.addupdate_scatter(hist_v, [idx16], ones16)
            return carry

        lax.fori_loop(0, epw // 64, body, 0)
        pltpu.sync_copy(hist_v, out_hbm.at[w])

    return deg_kernel


def _make_hop_kernel(np_nodes, n, epw, c):
    mesh = plsc.VectorSubcoreMesh(core_axis_name="c", subcore_axis_name="s")
    nchunk = epw // CHUNK
    rows_per_tile = np_nodes // NS
    drows = (n // NS) // 8 * 8
    drows_last = n - (NS - 1) * drows

    @functools.partial(
        pl.kernel,
        out_type=jax.ShapeDtypeStruct((NC, n, c), jnp.float32),
        mesh=mesh,
        compiler_params=pltpu.CompilerParams(needs_layout_passes=False),
        scratch_types=[
            pltpu.VMEM((nchunk, CHUNK), jnp.int32),
            pltpu.VMEM((nchunk, CHUNK), jnp.int32),
            [pltpu.VMEM((CHUNK, c), jnp.float32)] * NBUF,
            pltpu.VMEM((16, c), jnp.float32),
            pltpu.VMEM_SHARED((np_nodes, c), jnp.float32),
            [pltpu.SemaphoreType.DMA] * NBUF,
            [pltpu.SemaphoreType.DMA] * NBUF,
        ],
    )
    def hop_kernel(table_hbm, src_hbm, dst_hbm, zeros_hbm, out_hbm,
                   src_v, dst_v, bufs, zbuf, acc, gsems, ssems):
        ci = lax.axis_index("c")
        si = lax.axis_index("s")
        w = ci * NS + si
        cz = pltpu.async_copy(zeros_hbm, zbuf, ssems[0])
        cs = pltpu.async_copy(src_hbm.at[w], src_v, gsems[0])
        cd = pltpu.async_copy(dst_hbm.at[w], dst_v, gsems[1])
        cz.wait()
        base = si * rows_per_tile
        nz = rows_per_tile // 16
        for g in range(0, nz, 8):
            zcs = [pltpu.async_copy(
                       zbuf, acc.at[pl.ds(base + (g + u) * 16, 16)],
                       ssems[1])
                   for u in range(min(8, nz - g))]
            for zc in zcs:
                zc.wait()
        cs.wait()
        cd.wait()
        plsc.subcore_barrier()

        for i in range(NBUF - 1):
            pltpu.async_copy(table_hbm.at[src_v.at[i]], bufs[i], gsems[i])

        def ring(k, carry):
            j0 = k * NBUF
            for i in range(NBUF):
                j = j0 + i
                i3 = (i + NBUF - 1) % NBUF
                jn = j + NBUF - 1

                @pl.when(jnp.logical_and(jn < nchunk, j >= 1))
                def _():
                    pltpu.make_async_copy(
                        bufs[i3], acc.at[dst_v.at[j - 1]], ssems[i3]).wait()
                    pltpu.async_copy(table_hbm.at[src_v.at[jn]],
                                     bufs[i3], gsems[i3])

                @pl.when(jnp.logical_and(jn < nchunk, j < 1))
                def _():
                    pltpu.async_copy(table_hbm.at[src_v.at[jn]],
                                     bufs[i3], gsems[i3])

                pltpu.make_async_copy(table_hbm.at[src_v.at[j]],
                                      bufs[i], gsems[i]).wait()
                pltpu.async_copy(bufs[i], acc.at[dst_v.at[j]],
                                 ssems[i], add=True)
            return carry

        lax.fori_loop(0, nchunk // NBUF, ring, 0)
        for i in range(NBUF):
            j = nchunk - NBUF + i
            pltpu.make_async_copy(bufs[j % NBUF], acc.at[dst_v.at[j]],
                                  ssems[j % NBUF]).wait()
        plsc.subcore_barrier()
        @pl.when(si < NS - 1)
        def _():
            pltpu.sync_copy(acc.at[pl.ds(si * drows, drows)],
                            out_hbm.at[ci, pl.ds(si * drows, drows)])

        @pl.when(si == NS - 1)
        def _():
            pltpu.sync_copy(
                acc.at[pl.ds((NS - 1) * drows, drows_last)],
                out_hbm.at[ci, pl.ds((NS - 1) * drows, drows_last)])

    return hop_kernel


def _mm_norm_body(degp_ref, x_ref, w_ref, t0_ref, nrm_ref):
    i = pl.program_id(0)
    deg = jnp.maximum(jnp.sum(degp_ref[...], axis=0), 1.0)
    nrm = lax.rsqrt(deg)
    y = jnp.dot(x_ref[...], w_ref[...], preferred_element_type=jnp.float32)
    t0_ref[...] = y * nrm[:, None]
    nrm_ref[pl.ds(i * ROW_BLK, ROW_BLK)] = nrm


def _mid_body(p_ref, nrm_ref, o_ref):
    i = pl.program_id(0)
    nrm = nrm_ref[pl.ds(i * ROW_BLK, ROW_BLK)]
    o_ref[...] = (p_ref[0] + p_ref[1]) * (nrm * nrm)[:, None]


def _fin_body(p_ref, nrm_ref, b_ref, o_ref):
    i = pl.program_id(0)
    nrm = nrm_ref[pl.ds(i * ROW_BLK, ROW_BLK)]
    o_ref[...] = ((p_ref[0] + p_ref[1]) * nrm[:, None]
                  + b_ref[...][None, :])


def kernel(features, edge_index, W, b):
    n, f = features.shape
    c = W.shape[1]
    e = edge_index.shape[1]

    epw = _round_up(_round_up(e, NW) // NW, CHUNK * NBUF)
    e_pad = NW * epw
    np_nodes = _round_up(n + 1, ROW_BLK)
    rows_per_tile = np_nodes // NS
    nblk = np_nodes // ROW_BLK
    trash = n

    src = edge_index[0].astype(jnp.int32)
    dst = edge_index[1].astype(jnp.int32)
    n_pad_edges = e_pad - e
    pad_iota = jnp.arange(n_pad_edges, dtype=jnp.int32)
    n_trash = np_nodes - trash
    src_p = jnp.concatenate([src, pad_iota % n])
    dst_p = jnp.concatenate([dst, trash + pad_iota % n_trash])
    src3 = src_p.reshape(NW, epw // CHUNK, CHUNK)
    dst3 = dst_p.reshape(NW, epw // CHUNK, CHUNK)
    dst2 = dst_p.reshape(NW, epw)

    zeros_rows = jnp.zeros((16, c), jnp.float32)

    deg_kernel = _make_deg_kernel(np_nodes, epw)
    hop_kernel = _make_hop_kernel(np_nodes, n, epw, c)

    normk = pl.pallas_call(
        _mm_norm_body,
        grid=(nblk,),
        in_specs=[
            pl.BlockSpec((NW, ROW_BLK), lambda i: (0, i)),
            pl.BlockSpec((ROW_BLK, f), lambda i: (i, 0)),
            pl.BlockSpec((f, c), lambda i: (0, 0)),
        ],
        out_specs=[
            pl.BlockSpec((ROW_BLK, c), lambda i: (i, 0)),
            pl.BlockSpec((np_nodes,), lambda i: (0,)),
        ],
        out_shape=[
            jax.ShapeDtypeStruct((n, c), jnp.float32),
            jax.ShapeDtypeStruct((np_nodes,), jnp.float32),
        ],
    )

    midk = pl.pallas_call(
        _mid_body,
        grid=(nblk,),
        in_specs=[
            pl.BlockSpec((NC, ROW_BLK, c), lambda i: (0, i, 0)),
            pl.BlockSpec((np_nodes,), lambda i: (0,)),
        ],
        out_specs=pl.BlockSpec((ROW_BLK, c), lambda i: (i, 0)),
        out_shape=jax.ShapeDtypeStruct((n, c), jnp.float32),
    )

    fink = pl.pallas_call(
        _fin_body,
        grid=(nblk,),
        in_specs=[
            pl.BlockSpec((NC, ROW_BLK, c), lambda i: (0, i, 0)),
            pl.BlockSpec((np_nodes,), lambda i: (0,)),
            pl.BlockSpec((c,), lambda i: (0,)),
        ],
        out_specs=pl.BlockSpec((ROW_BLK, c), lambda i: (i, 0)),
        out_shape=jax.ShapeDtypeStruct((n, c), jnp.float32),
    )

    degpart = deg_kernel(dst2)
    t0, nrm = normk(degpart, features, W)
    p1 = hop_kernel(t0, src3, dst3, zeros_rows)
    t1 = midk(p1, nrm)
    p2 = hop_kernel(t1, src3, dst3, zeros_rows)
    return fink(p2, nrm, b)


# speedup vs baseline: 1.0633x; 1.0374x over previous
import functools

import jax
import jax.numpy as jnp
from jax import lax
from jax.experimental import pallas as pl
from jax.experimental.pallas import tpu as pltpu
from jax.experimental.pallas import tpu_sc as plsc

NC = 2
NS = 16
NW = NC * NS
CHUNK = 64
NBUF = 3
ROW_BLK = 2560


def _round_up(x, m):
    return (x + m - 1) // m * m


def _make_deg_kernel(np_nodes, epw):
    mesh = plsc.VectorSubcoreMesh(core_axis_name="c", subcore_axis_name="s")

    @functools.partial(
        pl.kernel,
        out_type=jax.ShapeDtypeStruct((NW, np_nodes), jnp.float32),
        mesh=mesh,
        compiler_params=pltpu.CompilerParams(needs_layout_passes=False),
        scratch_types=[
            pltpu.VMEM((epw,), jnp.int32),
            pltpu.VMEM((np_nodes,), jnp.float32),
        ],
    )
    def deg_kernel(dst_hbm, out_hbm, idx_v, hist_v):
        ci = lax.axis_index("c")
        si = lax.axis_index("s")
        w = ci * NS + si
        pltpu.sync_copy(dst_hbm.at[w], idx_v)
        z16 = jnp.zeros((16,), jnp.float32)

        def zbody(i, carry):
            hist_v[pl.ds(i * 16, 16)] = z16
            return carry

        lax.fori_loop(0, np_nodes // 16, zbody, 0)
        ones16 = jnp.ones((16,), jnp.float32)

        def body(i, carry):
            for u in range(4):
                idx16 = idx_v[pl.ds((i * 4 + u) * 16, 16)]
                plsc.addupdate_scatter(hist_v, [idx16], ones16)
            return carry

        lax.fori_loop(0, epw // 64, body, 0)
        pltpu.sync_copy(hist_v, out_hbm.at[w])

    return deg_kernel


def _make_hop_kernel(np_nodes, n, epw, c):
    mesh = plsc.VectorSubcoreMesh(core_axis_name="c", subcore_axis_name="s")
    nchunk = epw // CHUNK
    rows_per_tile = np_nodes // NS
    drows = (n // NS) // 8 * 8
    drows_last = n - (NS - 1) * drows

    @functools.partial(
        pl.kernel,
        out_type=jax.ShapeDtypeStruct((NC, n, c), jnp.float32),
        mesh=mesh,
        compiler_params=pltpu.CompilerParams(needs_layout_passes=False),
        scratch_types=[
            pltpu.VMEM((nchunk, CHUNK), jnp.int32),
            pltpu.VMEM((nchunk, CHUNK), jnp.int32),
            [pltpu.VMEM((CHUNK, c), jnp.float32)] * NBUF,
            pltpu.VMEM((16, c), jnp.float32),
            pltpu.VMEM_SHARED((np_nodes, c), jnp.float32),
            [pltpu.SemaphoreType.DMA] * NBUF,
            [pltpu.SemaphoreType.DMA] * NBUF,
        ],
    )
    def hop_kernel(table_hbm, src_hbm, dst_hbm, zeros_hbm, out_hbm,
                   src_v, dst_v, bufs, zbuf, acc, gsems, ssems):
        ci = lax.axis_index("c")
        si = lax.axis_index("s")
        w = ci * NS + si
        cz = pltpu.async_copy(zeros_hbm, zbuf, ssems[0])
        cs = pltpu.async_copy(src_hbm.at[w], src_v, gsems[0])
        cd = pltpu.async_copy(dst_hbm.at[w], dst_v, gsems[1])
        cz.wait()
        base = si * rows_per_tile
        nz = rows_per_tile // 16
        for g in range(0, nz, 8):
            zcs = [pltpu.async_copy(
                       zbuf, acc.at[pl.ds(base + (g + u) * 16, 16)],
                       ssems[1])
                   for u in range(min(8, nz - g))]
            for zc in zcs:
                zc.wait()
        cs.wait()
        cd.wait()
        plsc.subcore_barrier()

        for i in range(NBUF - 1):
            pltpu.async_copy(table_hbm.at[src_v.at[i]], bufs[i], gsems[i])

        def ring(k, carry):
            j0 = k * NBUF
            for i in range(NBUF):
                j = j0 + i
                i3 = (i + NBUF - 1) % NBUF
                jn = j + NBUF - 1

                @pl.when(jnp.logical_and(jn < nchunk, j >= 1))
                def _():
                    pltpu.make_async_copy(
                        bufs[i3], acc.at[dst_v.at[j - 1]], ssems[i3]).wait()
                    pltpu.async_copy(table_hbm.at[src_v.at[jn]],
                                     bufs[i3], gsems[i3])

                @pl.when(jnp.logical_and(jn < nchunk, j < 1))
                def _():
                    pltpu.async_copy(table_hbm.at[src_v.at[jn]],
                                     bufs[i3], gsems[i3])

                pltpu.make_async_copy(table_hbm.at[src_v.at[j]],
                                      bufs[i], gsems[i]).wait()
                pltpu.async_copy(bufs[i], acc.at[dst_v.at[j]],
                                 ssems[i], add=True)
            return carry

        lax.fori_loop(0, nchunk // NBUF, ring, 0)
        for i in range(NBUF):
            j = nchunk - NBUF + i
            pltpu.make_async_copy(bufs[j % NBUF], acc.at[dst_v.at[j]],
                                  ssems[j % NBUF]).wait()
        plsc.subcore_barrier()
        @pl.when(si < NS - 1)
        def _():
            pltpu.sync_copy(acc.at[pl.ds(si * drows, drows)],
                            out_hbm.at[ci, pl.ds(si * drows, drows)])

        @pl.when(si == NS - 1)
        def _():
            pltpu.sync_copy(
                acc.at[pl.ds((NS - 1) * drows, drows_last)],
                out_hbm.at[ci, pl.ds((NS - 1) * drows, drows_last)])

    return hop_kernel


def _mm_norm_body(degp_ref, x_ref, w_ref, t0_ref, nrm_ref):
    i = pl.program_id(0)
    deg = jnp.maximum(jnp.sum(degp_ref[...], axis=0), 1.0)
    nrm = lax.rsqrt(deg)
    y = jnp.dot(x_ref[...], w_ref[...], preferred_element_type=jnp.float32)
    t0_ref[...] = y * nrm[:, None]
    nrm_ref[pl.ds(i * ROW_BLK, ROW_BLK)] = nrm


def _mid_body(p_ref, nrm_ref, o_ref):
    i = pl.program_id(0)
    nrm = nrm_ref[pl.ds(i * ROW_BLK, ROW_BLK)]
    o_ref[...] = (p_ref[0] + p_ref[1]) * (nrm * nrm)[:, None]


def _fin_body(p_ref, nrm_ref, b_ref, o_ref):
    i = pl.program_id(0)
    nrm = nrm_ref[pl.ds(i * ROW_BLK, ROW_BLK)]
    o_ref[...] = ((p_ref[0] + p_ref[1]) * nrm[:, None]
                  + b_ref[...][None, :])


def kernel(features, edge_index, W, b):
    n, f = features.shape
    c = W.shape[1]
    e = edge_index.shape[1]

    epw = _round_up(_round_up(e, NW) // NW, CHUNK * NBUF)
    e_pad = NW * epw
    np_nodes = _round_up(n + 1, ROW_BLK)
    rows_per_tile = np_nodes // NS
    nblk = np_nodes // ROW_BLK
    trash = n

    src = edge_index[0].astype(jnp.int32)
    dst = edge_index[1].astype(jnp.int32)
    n_pad_edges = e_pad - e
    pad_iota = jnp.arange(n_pad_edges, dtype=jnp.int32)
    n_trash = np_nodes - trash
    src_p = jnp.concatenate([src, pad_iota % n])
    dst_p = jnp.concatenate([dst, trash + pad_iota % n_trash])
    src3 = src_p.reshape(NW, epw // CHUNK, CHUNK)
    dst3 = dst_p.reshape(NW, epw // CHUNK, CHUNK)
    dst2 = dst_p.reshape(NW, epw)

    zeros_rows = jnp.zeros((16, c), jnp.float32)

    deg_kernel = _make_deg_kernel(np_nodes, epw)
    hop_kernel = _make_hop_kernel(np_nodes, n, epw, c)

    normk = pl.pallas_call(
        _mm_norm_body,
        grid=(nblk,),
        in_specs=[
            pl.BlockSpec((NW, ROW_BLK), lambda i: (0, i)),
            pl.BlockSpec((ROW_BLK, f), lambda i: (i, 0)),
            pl.BlockSpec((f, c), lambda i: (0, 0)),
        ],
        out_specs=[
            pl.BlockSpec((ROW_BLK, c), lambda i: (i, 0)),
            pl.BlockSpec((np_nodes,), lambda i: (0,)),
        ],
        out_shape=[
            jax.ShapeDtypeStruct((n, c), jnp.float32),
            jax.ShapeDtypeStruct((np_nodes,), jnp.float32),
        ],
    )

    midk = pl.pallas_call(
        _mid_body,
        grid=(nblk,),
        in_specs=[
            pl.BlockSpec((NC, ROW_BLK, c), lambda i: (0, i, 0)),
            pl.BlockSpec((np_nodes,), lambda i: (0,)),
        ],
        out_specs=pl.BlockSpec((ROW_BLK, c), lambda i: (i, 0)),
        out_shape=jax.ShapeDtypeStruct((n, c), jnp.float32),
    )

    fink = pl.pallas_call(
        _fin_body,
        grid=(nblk,),
        in_specs=[
            pl.BlockSpec((NC, ROW_BLK, c), lambda i: (0, i, 0)),
            pl.BlockSpec((np_nodes,), lambda i: (0,)),
            pl.BlockSpec((c,), lambda i: (0,)),
        ],
        out_specs=pl.BlockSpec((ROW_BLK, c), lambda i: (i, 0)),
        out_shape=jax.ShapeDtypeStruct((n, c), jnp.float32),
    )

    degpart = deg_kernel(dst2)
    t0, nrm = normk(degpart, features, W)
    p1 = hop_kernel(t0, src3, dst3, zeros_rows)
    t1 = midk(p1, nrm)
    p2 = hop_kernel(t1, src3, dst3, zeros_rows)
    return fink(p2, nrm, b)


# speedup vs baseline: 1.0839x; 1.0194x over previous
import functools

import jax
import jax.numpy as jnp
from jax import lax
from jax.experimental import pallas as pl
from jax.experimental.pallas import tpu as pltpu
from jax.experimental.pallas import tpu_sc as plsc

NC = 2
NS = 16
NW = NC * NS
CHUNK = 64
NBUF = 3
ROW_BLK = 5120


def _round_up(x, m):
    return (x + m - 1) // m * m


def _make_deg_kernel(np_nodes, epw):
    mesh = plsc.VectorSubcoreMesh(core_axis_name="c", subcore_axis_name="s")

    @functools.partial(
        pl.kernel,
        out_type=jax.ShapeDtypeStruct((NW, np_nodes), jnp.float32),
        mesh=mesh,
        compiler_params=pltpu.CompilerParams(needs_layout_passes=False),
        scratch_types=[
            pltpu.VMEM((epw,), jnp.int32),
            pltpu.VMEM((np_nodes,), jnp.float32),
        ],
    )
    def deg_kernel(dst_hbm, out_hbm, idx_v, hist_v):
        ci = lax.axis_index("c")
        si = lax.axis_index("s")
        w = ci * NS + si
        pltpu.sync_copy(dst_hbm.at[w], idx_v)
        z16 = jnp.zeros((16,), jnp.float32)

        def zbody(i, carry):
            hist_v[pl.ds(i * 16, 16)] = z16
            return carry

        lax.fori_loop(0, np_nodes // 16, zbody, 0)
        ones16 = jnp.ones((16,), jnp.float32)

        def body(i, carry):
            for u in range(4):
                idx16 = idx_v[pl.ds((i * 4 + u) * 16, 16)]
                plsc.addupdate_scatter(hist_v, [idx16], ones16)
            return carry

        lax.fori_loop(0, epw // 64, body, 0)
        pltpu.sync_copy(hist_v, out_hbm.at[w])

    return deg_kernel


def _make_hop_kernel(np_nodes, n, epw, c):
    mesh = plsc.VectorSubcoreMesh(core_axis_name="c", subcore_axis_name="s")
    nchunk = epw // CHUNK
    rows_per_tile = np_nodes // NS
    drows = (n // NS) // 8 * 8
    drows_last = n - (NS - 1) * drows

    @functools.partial(
        pl.kernel,
        out_type=jax.ShapeDtypeStruct((NC, n, c), jnp.float32),
        mesh=mesh,
        compiler_params=pltpu.CompilerParams(needs_layout_passes=False),
        scratch_types=[
            pltpu.VMEM((nchunk, CHUNK), jnp.int32),
            pltpu.VMEM((nchunk, CHUNK), jnp.int32),
            [pltpu.VMEM((CHUNK, c), jnp.float32)] * NBUF,
            pltpu.VMEM((16, c), jnp.float32),
            pltpu.VMEM_SHARED((np_nodes, c), jnp.float32),
            [pltpu.SemaphoreType.DMA] * NBUF,
            [pltpu.SemaphoreType.DMA] * NBUF,
        ],
    )
    def hop_kernel(table_hbm, src_hbm, dst_hbm, zeros_hbm, out_hbm,
                   src_v, dst_v, bufs, zbuf, acc, gsems, ssems):
        ci = lax.axis_index("c")
        si = lax.axis_index("s")
        w = ci * NS + si
        cz = pltpu.async_copy(zeros_hbm, zbuf, ssems[0])
        cs = pltpu.async_copy(src_hbm.at[w], src_v, gsems[0])
        cd = pltpu.async_copy(dst_hbm.at[w], dst_v, gsems[1])
        cz.wait()
        base = si * rows_per_tile
        nz = rows_per_tile // 16
        for g in range(0, nz, 8):
            zcs = [pltpu.async_copy(
                       zbuf, acc.at[pl.ds(base + (g + u) * 16, 16)],
                       ssems[1])
                   for u in range(min(8, nz - g))]
            for zc in zcs:
                zc.wait()
        cs.wait()
        cd.wait()
        plsc.subcore_barrier()

        for i in range(NBUF - 1):
            pltpu.async_copy(table_hbm.at[src_v.at[i]], bufs[i], gsems[i])

        def ring(k, carry):
            j0 = k * NBUF
            for i in range(NBUF):
                j = j0 + i
                i3 = (i + NBUF - 1) % NBUF
                jn = j + NBUF - 1

                @pl.when(jnp.logical_and(jn < nchunk, j >= 1))
                def _():
                    pltpu.make_async_copy(
                        bufs[i3], acc.at[dst_v.at[j - 1]], ssems[i3]).wait()
                    pltpu.async_copy(table_hbm.at[src_v.at[jn]],
                                     bufs[i3], gsems[i3])

                @pl.when(jnp.logical_and(jn < nchunk, j < 1))
                def _():
                    pltpu.async_copy(table_hbm.at[src_v.at[jn]],
                                     bufs[i3], gsems[i3])

                pltpu.make_async_copy(table_hbm.at[src_v.at[j]],
                                      bufs[i], gsems[i]).wait()
                pltpu.async_copy(bufs[i], acc.at[dst_v.at[j]],
                                 ssems[i], add=True)
            return carry

        lax.fori_loop(0, nchunk // NBUF, ring, 0)
        for i in range(NBUF):
            j = nchunk - NBUF + i
            pltpu.make_async_copy(bufs[j % NBUF], acc.at[dst_v.at[j]],
                                  ssems[j % NBUF]).wait()
        plsc.subcore_barrier()
        @pl.when(si < NS - 1)
        def _():
            pltpu.sync_copy(acc.at[pl.ds(si * drows, drows)],
                            out_hbm.at[ci, pl.ds(si * drows, drows)])

        @pl.when(si == NS - 1)
        def _():
            pltpu.sync_copy(
                acc.at[pl.ds((NS - 1) * drows, drows_last)],
                out_hbm.at[ci, pl.ds((NS - 1) * drows, drows_last)])

    return hop_kernel


def _mm_norm_body(degp_ref, x_ref, w_ref, t0_ref, nrm_ref):
    i = pl.program_id(0)
    deg = jnp.maximum(jnp.sum(degp_ref[...], axis=0), 1.0)
    nrm = lax.rsqrt(deg)
    y = jnp.dot(x_ref[...], w_ref[...], preferred_element_type=jnp.float32)
    t0_ref[...] = y * nrm[:, None]
    nrm_ref[pl.ds(i * ROW_BLK, ROW_BLK)] = nrm


def _mid_body(p_ref, nrm_ref, o_ref):
    i = pl.program_id(0)
    nrm = nrm_ref[pl.ds(i * ROW_BLK, ROW_BLK)]
    o_ref[...] = (p_ref[0] + p_ref[1]) * (nrm * nrm)[:, None]


def _fin_body(p_ref, nrm_ref, b_ref, o_ref):
    i = pl.program_id(0)
    nrm = nrm_ref[pl.ds(i * ROW_BLK, ROW_BLK)]
    o_ref[...] = ((p_ref[0] + p_ref[1]) * nrm[:, None]
                  + b_ref[...][None, :])


def kernel(features, edge_index, W, b):
    n, f = features.shape
    c = W.shape[1]
    e = edge_index.shape[1]

    epw = _round_up(_round_up(e, NW) // NW, CHUNK * NBUF)
    e_pad = NW * epw
    np_nodes = _round_up(n + 1, ROW_BLK)
    rows_per_tile = np_nodes // NS
    nblk = np_nodes // ROW_BLK
    trash = n

    src = edge_index[0].astype(jnp.int32)
    dst = edge_index[1].astype(jnp.int32)
    n_pad_edges = e_pad - e
    pad_iota = jnp.arange(n_pad_edges, dtype=jnp.int32)
    n_trash = np_nodes - trash
    src_p = jnp.concatenate([src, pad_iota % n])
    dst_p = jnp.concatenate([dst, trash + pad_iota % n_trash])
    src3 = src_p.reshape(NW, epw // CHUNK, CHUNK)
    dst3 = dst_p.reshape(NW, epw // CHUNK, CHUNK)
    dst2 = dst_p.reshape(NW, epw)

    zeros_rows = jnp.zeros((16, c), jnp.float32)

    deg_kernel = _make_deg_kernel(np_nodes, epw)
    hop_kernel = _make_hop_kernel(np_nodes, n, epw, c)

    normk = pl.pallas_call(
        _mm_norm_body,
        grid=(nblk,),
        in_specs=[
            pl.BlockSpec((NW, ROW_BLK), lambda i: (0, i)),
            pl.BlockSpec((ROW_BLK, f), lambda i: (i, 0)),
            pl.BlockSpec((f, c), lambda i: (0, 0)),
        ],
        out_specs=[
            pl.BlockSpec((ROW_BLK, c), lambda i: (i, 0)),
            pl.BlockSpec((np_nodes,), lambda i: (0,)),
        ],
        out_shape=[
            jax.ShapeDtypeStruct((n, c), jnp.float32),
            jax.ShapeDtypeStruct((np_nodes,), jnp.float32),
        ],
    )

    midk = pl.pallas_call(
        _mid_body,
        grid=(nblk,),
        in_specs=[
            pl.BlockSpec((NC, ROW_BLK, c), lambda i: (0, i, 0)),
            pl.BlockSpec((np_nodes,), lambda i: (0,)),
        ],
        out_specs=pl.BlockSpec((ROW_BLK, c), lambda i: (i, 0)),
        out_shape=jax.ShapeDtypeStruct((n, c), jnp.float32),
    )

    fink = pl.pallas_call(
        _fin_body,
        grid=(nblk,),
        in_specs=[
            pl.BlockSpec((NC, ROW_BLK, c), lambda i: (0, i, 0)),
            pl.BlockSpec((np_nodes,), lambda i: (0,)),
            pl.BlockSpec((c,), lambda i: (0,)),
        ],
        out_specs=pl.BlockSpec((ROW_BLK, c), lambda i: (i, 0)),
        out_shape=jax.ShapeDtypeStruct((n, c), jnp.float32),
    )

    degpart = deg_kernel(dst2)
    t0, nrm = normk(degpart, features, W)
    p1 = hop_kernel(t0, src3, dst3, zeros_rows)
    t1 = midk(p1, nrm)
    p2 = hop_kernel(t1, src3, dst3, zeros_rows)
    return fink(p2, nrm, b)
